# baseline (speedup 1.0000x reference)
import functools

import jax
import jax.numpy as jnp
from jax import lax
from jax.experimental import pallas as pl
from jax.experimental.pallas import tpu as pltpu
from jax.experimental.pallas import tpu_sc as plsc

N = 10000
IN = 32
H = 128
OUT = N + 1
E = 160000

NC = 2
NS = 16
NW = NC * NS

NPAD = 10240
F0 = 128
ONES_COL = IN + 2
OUTP = 10240

CH = 128
NECH = E // CH
RPT = NPAD // NS

_PREC = jax.lax.Precision.HIGHEST


def _mesh():
    return plsc.VectorSubcoreMesh(
        core_axis_name="c", subcore_axis_name="s", num_cores=NC, num_subcores=NS
    )


def _make_agg(F):
    @functools.partial(
        pl.kernel,
        mesh=_mesh(),
        out_type=jax.ShapeDtypeStruct((NC, NPAD, F), jnp.float32),
        scratch_types=[
            pltpu.VMEM((CH,), jnp.int32),
            pltpu.VMEM((CH,), jnp.int32),
            pltpu.VMEM((CH, F), jnp.float32),
            pltpu.VMEM_SHARED((NPAD, F), jnp.float32),
            pltpu.SemaphoreType.DMA,
        ],
    )
    def agg(h_hbm, src_hbm, dst_hbm, zrows_hbm, out_hbm, idx_s, idx_d, rows, acc, sem):
        c = lax.axis_index("c")
        s = lax.axis_index("s")
        wid = s * NC + c
        row0 = s * RPT
        pltpu.sync_copy(zrows_hbm, acc.at[pl.ds(row0, RPT)])
        plsc.subcore_barrier()

        def chunk(k, carry):
            cid = wid + k * NW

            @pl.when(cid < NECH)
            def _():
                off = cid * CH
                pltpu.sync_copy(src_hbm.at[pl.ds(off, CH)], idx_s)
                pltpu.sync_copy(dst_hbm.at[pl.ds(off, CH)], idx_d)
                pltpu.async_copy(h_hbm.at[idx_s], rows, sem).wait()
                pltpu.sync_copy(rows, acc.at[idx_d], add=True)

            return carry

        lax.fori_loop(0, (NECH + NW - 1) // NW, chunk, 0)
        plsc.subcore_barrier()
        pltpu.sync_copy(acc.at[pl.ds(row0, RPT)], out_hbm.at[c, pl.ds(row0, RPT)])

    return agg


_agg = _make_agg(H)


_RT = 1024


def _layer1_body(p_ref, h_ref, wl_ref, wr_ref, b_ref, out_ref, invd_ref):
    tot = p_ref[0] + p_ref[1]
    deg = tot[:, ONES_COL]
    invd = 1.0 / jnp.maximum(deg, 1.0)
    mean = tot * invd[:, None]
    acc = (
        jnp.dot(mean, wl_ref[...], precision=_PREC, preferred_element_type=jnp.float32)
        + b_ref[...]
        + jnp.dot(h_ref[...], wr_ref[...], precision=_PREC, preferred_element_type=jnp.float32)
    )
    i = pl.program_id(0)
    rows = i * _RT + lax.broadcasted_iota(jnp.int32, (_RT, 1), 0)
    out_ref[...] = jnp.where(rows < N, jnp.maximum(acc, 0.0), 0.0)
    invd_ref[...] = invd


def _layerN_body(p_ref, h_ref, invd_ref, wl_ref, wr_ref, b_ref, out_ref):
    invd = invd_ref[...]
    mean = (p_ref[0] + p_ref[1]) * invd[:, None]
    acc = (
        jnp.dot(mean, wl_ref[...], precision=_PREC, preferred_element_type=jnp.float32)
        + b_ref[...]
        + jnp.dot(h_ref[...], wr_ref[...], precision=_PREC, preferred_element_type=jnp.float32)
    )
    i = pl.program_id(0)
    rows = i * _RT + lax.broadcasted_iota(jnp.int32, (_RT, 1), 0)
    out_ref[...] = jnp.where(rows < N, jnp.maximum(acc, 0.0), 0.0)


def _tc_layer1(p, h, wl, wr, b):
    F = h.shape[1]
    return pl.pallas_call(
        _layer1_body,
        grid=(NPAD // _RT,),
        in_specs=[
            pl.BlockSpec((NC, _RT, F), lambda i: (0, i, 0)),
            pl.BlockSpec((_RT, F), lambda i: (i, 0)),
            pl.BlockSpec((F, H), lambda i: (0, 0)),
            pl.BlockSpec((F, H), lambda i: (0, 0)),
            pl.BlockSpec((1, H), lambda i: (0, 0)),
        ],
        out_specs=[
            pl.BlockSpec((_RT, H), lambda i: (i, 0)),
            pl.BlockSpec((_RT,), lambda i: (i,)),
        ],
        out_shape=[
            jax.ShapeDtypeStruct((NPAD, H), jnp.float32),
            jax.ShapeDtypeStruct((NPAD,), jnp.float32),
        ],
    )(p, h, wl, wr, b)


def _tc_layerN(p, h, invd, wl, wr, b):
    return pl.pallas_call(
        _layerN_body,
        grid=(NPAD // _RT,),
        in_specs=[
            pl.BlockSpec((NC, _RT, H), lambda i: (0, i, 0)),
            pl.BlockSpec((_RT, H), lambda i: (i, 0)),
            pl.BlockSpec((_RT,), lambda i: (i,)),
            pl.BlockSpec((H, H), lambda i: (0, 0)),
            pl.BlockSpec((H, H), lambda i: (0, 0)),
            pl.BlockSpec((1, H), lambda i: (0, 0)),
        ],
        out_specs=pl.BlockSpec((_RT, H), lambda i: (i, 0)),
        out_shape=jax.ShapeDtypeStruct((NPAD, H), jnp.float32),
    )(p, h, invd, wl, wr, b)


_RL = 1024
_CL = 1280


def _logits_body(h_ref, wfc_ref, bfc_ref, logits_ref, rmax_ref):
    j = pl.program_id(1)
    l = (
        jnp.dot(h_ref[...], wfc_ref[...], precision=jax.lax.Precision.DEFAULT,
                preferred_element_type=jnp.float32)
        + bfc_ref[...]
    )
    logits_ref[...] = l
    bm = jnp.full((16,), jnp.max(l), jnp.float32)
    first = (pl.program_id(0) == 0) & (j == 0)

    @pl.when(first)
    def _():
        rmax_ref[...] = bm

    @pl.when(jnp.logical_not(first))
    def _():
        rmax_ref[...] = jnp.maximum(rmax_ref[...], bm)


def _tc_logits(h3, wfc, bfc):
    return pl.pallas_call(
        _logits_body,
        grid=(pl.cdiv(N, _RL), OUTP // _CL),
        in_specs=[
            pl.BlockSpec((_RL, H), lambda i, j: (i, 0)),
            pl.BlockSpec((H, _CL), lambda i, j: (0, j)),
            pl.BlockSpec((1, _CL), lambda i, j: (0, j)),
        ],
        out_specs=[
            pl.BlockSpec((_RL, _CL), lambda i, j: (i, j)),
            pl.BlockSpec((16,), lambda i, j: (0,)),
        ],
        out_shape=[
            jax.ShapeDtypeStruct((N, OUTP), jnp.float32),
            jax.ShapeDtypeStruct((16,), jnp.float32),
        ],
    )(h3, wfc, bfc)


HROW = N // 2
HFLAT = HROW * OUTP
ZCH = 16384
NZCH = HFLAT // ZCH


NKT = (NECH + NS - 1) // NS
EPM = NKT * CH
EPADM = EPM * NS
NZT = (NZCH + NS - 1) // NS


@functools.partial(
    pl.kernel,
    mesh=_mesh(),
    out_type=jax.ShapeDtypeStruct((N * OUTP,), jnp.float32),
    scratch_types=[
        pltpu.VMEM((ZCH,), jnp.float32),
        pltpu.VMEM((EPM,), jnp.int32),
        pltpu.VMEM((EPM,), jnp.int32),
        pltpu.VMEM((NKT, CH), jnp.int32),
        pltpu.VMEM((NKT, CH), jnp.int32),
        pltpu.VMEM((16,), jnp.float32),
        pltpu.VMEM((NKT, CH), jnp.float32),
        pltpu.VMEM((NKT, CH), jnp.float32),
        pltpu.SemaphoreType.DMA,
        pltpu.SemaphoreType.DMA,
    ],
)
def _sc_mask(logits_hbm, gmax_hbm, src_hbm, dst_hbm, z_hbm,
             zb, is1, id1, fl2, fz2, gm_v, lv2, vo2, gsem, ssem):
    c = lax.axis_index("c")
    s = lax.axis_index("s")
    zeros16 = jnp.zeros((16,), jnp.float32)
    pltpu.sync_copy(gmax_hbm, gm_v)

    def zfill(i, carry):
        zb[pl.ds(i * 16, 16)] = zeros16
        return carry

    with jax.named_scope("mask_zfill"):
        lax.fori_loop(0, ZCH // 16, zfill, 0)

    cbase = c * HFLAT
    lo = c * HROW
    hi = lo + HROW
    lane = lax.iota(jnp.int32, 16)
    base = s * EPM

    pltpu.sync_copy(src_hbm.at[pl.ds(base, EPM)], is1)
    pltpu.sync_copy(dst_hbm.at[pl.ds(base, EPM)], id1)

    def flts(k, carry):
        for t in range(CH // 16):
            sl = pl.ds(t * 16, 16)
            el = pl.ds(k * CH + t * 16, 16)
            vs = is1[el]
            vd = id1[el]
            fl2[k, sl] = vs * OUTP + vd
            inb = (vs >= lo) & (vs < hi)
            tr = lo + jax.lax.rem(base + k * CH + t * 16 + lane, HROW)
            fz2[k, sl] = jnp.where(inb, vs * OUTP + vd, tr * OUTP + N)
        return carry

    with jax.named_scope("mask_flt"):
        lax.fori_loop(0, NKT, flts, 0)

    def gfire(k, carry):
        pltpu.async_copy(logits_hbm.at[fl2.at[k]], lv2.at[k], gsem)
        return carry

    with jax.named_scope("mask_gfire"):
        lax.fori_loop(0, NKT, gfire, 0)

    def zchunk(it, carry):
        zcid = s + it * NS

        @pl.when(zcid < NZCH)
        def _():
            pltpu.sync_copy(zb, z_hbm.at[pl.ds(cbase + zcid * ZCH, ZCH)])

        return carry

    with jax.named_scope("mask_zero_hbm"):
        lax.fori_loop(0, NZT, zchunk, 0)

    def gdrain(k, carry):
        pltpu.make_async_copy(logits_hbm.at[fl2.at[0]], lv2.at[0], gsem).wait()
        return carry

    with jax.named_scope("mask_gdrain"):
        lax.fori_loop(0, NKT, gdrain, 0)
    gm = gm_v[...]

    def expk(k, carry):
        for t in range(CH // 16):
            sl = pl.ds(t * 16, 16)
            el = pl.ds(k * CH + t * 16, 16)
            vs = is1[el]
            inb = (vs >= lo) & (vs < hi)
            vo2[k, sl] = jnp.where(inb, jnp.exp(lv2[k, sl] - gm), 0.0)
        return carry

    with jax.named_scope("mask_exp"):
        lax.fori_loop(0, NKT, expk, 0)

    plsc.subcore_barrier()

    def sfire(k, carry):
        pltpu.async_copy(vo2.at[k], z_hbm.at[fz2.at[k]], ssem)
        return carry

    def sdrain(k, carry):
        pltpu.make_async_copy(vo2.at[0], z_hbm.at[fz2.at[0]], ssem).wait()
        return carry

    with jax.named_scope("mask_scatter"):
        lax.fori_loop(0, NKT, sfire, 0)
        lax.fori_loop(0, NKT, sdrain, 0)


_R6 = 200


def _renorm_body(s_ref, out_ref):
    blk = s_ref[...].reshape(_R6, OUTP)
    rs = jnp.sum(blk, axis=1)
    out_ref[...] = (blk * (1.0 / jnp.maximum(rs, 1e-30))[:, None])[:, :OUT]


def _tc_renorm(S_flat):
    return pl.pallas_call(
        _renorm_body,
        grid=(N // _R6,),
        in_specs=[pl.BlockSpec((_R6 * OUTP,), lambda i: (i,))],
        out_specs=pl.BlockSpec((_R6, OUT), lambda i: (i, 0)),
        out_shape=jax.ShapeDtypeStruct((N, OUT), jnp.float32),
    )(S_flat)


def kernel(x, edge_index, current_node, target_node,
           W1l, b1, W1r, W2l, b2, W2r, W3l, b3, W3r, Wfc, bfc):
    f32 = jnp.float32

    h0 = jnp.zeros((NPAD, F0), f32)
    h0 = h0.at[:N, :IN].set(x)
    h0 = h0.at[current_node, IN].set(1.0)
    h0 = h0.at[target_node, IN + 1].set(1.0)
    h0 = h0.at[:N, ONES_COL].set(1.0)

    w1l = jnp.zeros((F0, H), f32).at[: IN + 2].set(W1l)
    w1r = jnp.zeros((F0, H), f32).at[: IN + 2].set(W1r)
    b1r = b1.reshape(1, H)
    b2r = b2.reshape(1, H)
    b3r = b3.reshape(1, H)

    wfc = jnp.zeros((H, OUTP), f32).at[:, :OUT].set(Wfc)
    bfcp = jnp.full((OUTP,), -1e30, f32).at[:OUT].set(bfc).reshape(1, OUTP)

    src = edge_index[0]
    dst = edge_index[1]

    z128 = jnp.zeros((RPT, H), f32)

    p1 = _agg(h0, src, dst, z128)
    h1, invd = _tc_layer1(p1, h0, w1l, w1r, b1r)
    p2 = _agg(h1, src, dst, z128)
    h2 = _tc_layerN(p2, h1, invd, W2l, W2r, b2r)
    p3 = _agg(h2, src, dst, z128)
    h3 = _tc_layerN(p3, h2, invd, W3l, W3r, b3r)

    logits, gmax = _tc_logits(h3, wfc, bfcp)

    pad = EPADM - E
    src_k = jnp.concatenate([src, jnp.broadcast_to(src[0], (pad,))])
    dst_k = jnp.concatenate([dst, jnp.broadcast_to(dst[0], (pad,))])

    S_flat = _sc_mask(logits.reshape(-1), gmax, src_k, dst_k)

    return _tc_renorm(S_flat)


# speedup vs baseline: 1.0158x; 1.0158x over previous
import functools

import jax
import jax.numpy as jnp
from jax import lax
from jax.experimental import pallas as pl
from jax.experimental.pallas import tpu as pltpu
from jax.experimental.pallas import tpu_sc as plsc

N = 10000
IN = 32
H = 128
OUT = N + 1
E = 160000

NC = 2
NS = 16
NW = NC * NS

NPAD = 10240
F0 = 128
ONES_COL = IN + 2
OUTP = 10240

CH = 128
NECH = E // CH
RPT = NPAD // NS

_PREC = jax.lax.Precision.HIGHEST


def _mesh():
    return plsc.VectorSubcoreMesh(
        core_axis_name="c", subcore_axis_name="s", num_cores=NC, num_subcores=NS
    )


def _make_agg(F):
    @functools.partial(
        pl.kernel,
        mesh=_mesh(),
        out_type=jax.ShapeDtypeStruct((NC, NPAD, F), jnp.float32),
        scratch_types=[
            pltpu.VMEM((CH,), jnp.int32),
            pltpu.VMEM((CH,), jnp.int32),
            pltpu.VMEM((CH, F), jnp.float32),
            pltpu.VMEM_SHARED((NPAD, F), jnp.float32),
            pltpu.SemaphoreType.DMA,
        ],
    )
    def agg(h_hbm, src_hbm, dst_hbm, zrows_hbm, out_hbm, idx_s, idx_d, rows, acc, sem):
        c = lax.axis_index("c")
        s = lax.axis_index("s")
        wid = s * NC + c
        row0 = s * RPT
        pltpu.sync_copy(zrows_hbm, acc.at[pl.ds(row0, RPT)])
        plsc.subcore_barrier()

        def chunk(k, carry):
            cid = wid + k * NW

            @pl.when(cid < NECH)
            def _():
                off = cid * CH
                pltpu.sync_copy(src_hbm.at[pl.ds(off, CH)], idx_s)
                pltpu.sync_copy(dst_hbm.at[pl.ds(off, CH)], idx_d)
                pltpu.async_copy(h_hbm.at[idx_s], rows, sem).wait()
                pltpu.sync_copy(rows, acc.at[idx_d], add=True)

            return carry

        lax.fori_loop(0, (NECH + NW - 1) // NW, chunk, 0)
        plsc.subcore_barrier()
        pltpu.sync_copy(acc.at[pl.ds(row0, RPT)], out_hbm.at[c, pl.ds(row0, RPT)])

    return agg


_agg = _make_agg(H)


_RT = 1024


def _layer1_body(p_ref, h_ref, wl_ref, wr_ref, b_ref, out_ref, invd_ref):
    tot = p_ref[0] + p_ref[1]
    deg = tot[:, ONES_COL]
    invd = 1.0 / jnp.maximum(deg, 1.0)
    mean = tot * invd[:, None]
    acc = (
        jnp.dot(mean, wl_ref[...], precision=_PREC, preferred_element_type=jnp.float32)
        + b_ref[...]
        + jnp.dot(h_ref[...], wr_ref[...], precision=_PREC, preferred_element_type=jnp.float32)
    )
    i = pl.program_id(0)
    rows = i * _RT + lax.broadcasted_iota(jnp.int32, (_RT, 1), 0)
    out_ref[...] = jnp.where(rows < N, jnp.maximum(acc, 0.0), 0.0)
    invd_ref[...] = invd


def _layerN_body(p_ref, h_ref, invd_ref, wl_ref, wr_ref, b_ref, out_ref):
    invd = invd_ref[...]
    mean = (p_ref[0] + p_ref[1]) * invd[:, None]
    acc = (
        jnp.dot(mean, wl_ref[...], precision=_PREC, preferred_element_type=jnp.float32)
        + b_ref[...]
        + jnp.dot(h_ref[...], wr_ref[...], precision=_PREC, preferred_element_type=jnp.float32)
    )
    i = pl.program_id(0)
    rows = i * _RT + lax.broadcasted_iota(jnp.int32, (_RT, 1), 0)
    out_ref[...] = jnp.where(rows < N, jnp.maximum(acc, 0.0), 0.0)


def _tc_layer1(p, h, wl, wr, b):
    F = h.shape[1]
    return pl.pallas_call(
        _layer1_body,
        grid=(NPAD // _RT,),
        in_specs=[
            pl.BlockSpec((NC, _RT, F), lambda i: (0, i, 0)),
            pl.BlockSpec((_RT, F), lambda i: (i, 0)),
            pl.BlockSpec((F, H), lambda i: (0, 0)),
            pl.BlockSpec((F, H), lambda i: (0, 0)),
            pl.BlockSpec((1, H), lambda i: (0, 0)),
        ],
        out_specs=[
            pl.BlockSpec((_RT, H), lambda i: (i, 0)),
            pl.BlockSpec((_RT,), lambda i: (i,)),
        ],
        out_shape=[
            jax.ShapeDtypeStruct((NPAD, H), jnp.float32),
            jax.ShapeDtypeStruct((NPAD,), jnp.float32),
        ],
    )(p, h, wl, wr, b)


def _tc_layerN(p, h, invd, wl, wr, b):
    return pl.pallas_call(
        _layerN_body,
        grid=(NPAD // _RT,),
        in_specs=[
            pl.BlockSpec((NC, _RT, H), lambda i: (0, i, 0)),
            pl.BlockSpec((_RT, H), lambda i: (i, 0)),
            pl.BlockSpec((_RT,), lambda i: (i,)),
            pl.BlockSpec((H, H), lambda i: (0, 0)),
            pl.BlockSpec((H, H), lambda i: (0, 0)),
            pl.BlockSpec((1, H), lambda i: (0, 0)),
        ],
        out_specs=pl.BlockSpec((_RT, H), lambda i: (i, 0)),
        out_shape=jax.ShapeDtypeStruct((NPAD, H), jnp.float32),
    )(p, h, invd, wl, wr, b)


_RL = 1024
_CL = 1280


def _logits_body(h_ref, wfc_ref, bfc_ref, logits_ref, rmax_ref):
    j = pl.program_id(1)
    l = (
        jnp.dot(h_ref[...], wfc_ref[...], precision=jax.lax.Precision.DEFAULT,
                preferred_element_type=jnp.float32)
        + bfc_ref[...]
    )
    logits_ref[...] = l
    bm = jnp.full((16,), jnp.max(l), jnp.float32)
    first = (pl.program_id(0) == 0) & (j == 0)

    @pl.when(first)
    def _():
        rmax_ref[...] = bm

    @pl.when(jnp.logical_not(first))
    def _():
        rmax_ref[...] = jnp.maximum(rmax_ref[...], bm)


def _tc_logits(h3, wfc, bfc):
    return pl.pallas_call(
        _logits_body,
        grid=(pl.cdiv(N, _RL), OUTP // _CL),
        in_specs=[
            pl.BlockSpec((_RL, H), lambda i, j: (i, 0)),
            pl.BlockSpec((H, _CL), lambda i, j: (0, j)),
            pl.BlockSpec((1, _CL), lambda i, j: (0, j)),
        ],
        out_specs=[
            pl.BlockSpec((_RL, _CL), lambda i, j: (i, j)),
            pl.BlockSpec((16,), lambda i, j: (0,)),
        ],
        out_shape=[
            jax.ShapeDtypeStruct((N, OUTP), jnp.float32),
            jax.ShapeDtypeStruct((16,), jnp.float32),
        ],
    )(h3, wfc, bfc)


HROW = N // 2
HFLAT = HROW * OUTP
ZCH = 16384
NZCH = HFLAT // ZCH


NKT = (NECH + NS - 1) // NS
EPM = NKT * CH
EPADM = EPM * NS
NZT = (NZCH + NS - 1) // NS


@functools.partial(
    pl.kernel,
    mesh=_mesh(),
    out_type=jax.ShapeDtypeStruct((N * OUTP,), jnp.float32),
    scratch_types=[
        pltpu.VMEM((ZCH,), jnp.float32),
        pltpu.VMEM((CH,), jnp.int32),
        pltpu.VMEM((CH,), jnp.int32),
        pltpu.VMEM((CH,), jnp.int32),
        pltpu.VMEM((CH,), jnp.int32),
        pltpu.VMEM((16,), jnp.float32),
        pltpu.VMEM((CH,), jnp.float32),
        pltpu.VMEM((CH,), jnp.float32),
        pltpu.SemaphoreType.DMA,
    ],
)
def _sc_mask(logits_hbm, gmax_hbm, src_hbm, dst_hbm, z_hbm,
             zb, idx_s, idx_d, flt_l, flt_z, gm_v, lv, vo, sem):
    c = lax.axis_index("c")
    s = lax.axis_index("s")
    zeros16 = jnp.zeros((16,), jnp.float32)
    pltpu.sync_copy(gmax_hbm, gm_v)

    def zfill(i, carry):
        zb[pl.ds(i * 16, 16)] = zeros16
        return carry

    with jax.named_scope("mask_zfill"):
        lax.fori_loop(0, ZCH // 16, zfill, 0)

    cbase = c * HFLAT

    def zchunk(it, carry):
        zcid = s + it * NS

        @pl.when(zcid < NZCH)
        def _():
            pltpu.sync_copy(zb, z_hbm.at[pl.ds(cbase + zcid * ZCH, ZCH)])

        return carry

    with jax.named_scope("mask_zero_hbm"):
        lax.fori_loop(0, NZT, zchunk, 0)

    plsc.subcore_barrier()

    lo = c * HROW
    hi = lo + HROW
    lane = lax.iota(jnp.int32, 16)
    base = s * EPM
    gm = gm_v[...]

    def chunk(k, carry):
        off = base + k * CH
        pltpu.sync_copy(src_hbm.at[pl.ds(off, CH)], idx_s)
        pltpu.sync_copy(dst_hbm.at[pl.ds(off, CH)], idx_d)
        for t in range(CH // 16):
            sl = pl.ds(t * 16, 16)
            vs = idx_s[sl]
            vd = idx_d[sl]
            flt_l[sl] = vs * OUTP + vd
            inb = (vs >= lo) & (vs < hi)
            tr = lo + jax.lax.rem(off + t * 16 + lane, HROW)
            flt_z[sl] = jnp.where(inb, vs * OUTP + vd, tr * OUTP + N)
        pltpu.async_copy(logits_hbm.at[flt_l], lv, sem).wait()
        for t in range(CH // 16):
            sl = pl.ds(t * 16, 16)
            vs = idx_s[sl]
            inb = (vs >= lo) & (vs < hi)
            vo[sl] = jnp.where(inb, jnp.exp(lv[sl] - gm), 0.0)
        pltpu.async_copy(vo, z_hbm.at[flt_z], sem).wait()
        return carry

    with jax.named_scope("mask_edges"):
        lax.fori_loop(0, NKT, chunk, 0)


_R6 = 200


def _renorm_body(s_ref, out_ref):
    blk = s_ref[...].reshape(_R6, OUTP)
    rs = jnp.sum(blk, axis=1)
    out_ref[...] = (blk * (1.0 / jnp.maximum(rs, 1e-30))[:, None])[:, :OUT]


def _tc_renorm(S_flat):
    return pl.pallas_call(
        _renorm_body,
        grid=(N // _R6,),
        in_specs=[pl.BlockSpec((_R6 * OUTP,), lambda i: (i,))],
        out_specs=pl.BlockSpec((_R6, OUT), lambda i: (i, 0)),
        out_shape=jax.ShapeDtypeStruct((N, OUT), jnp.float32),
    )(S_flat)


def kernel(x, edge_index, current_node, target_node,
           W1l, b1, W1r, W2l, b2, W2r, W3l, b3, W3r, Wfc, bfc):
    f32 = jnp.float32

    h0 = jnp.zeros((NPAD, F0), f32)
    h0 = h0.at[:N, :IN].set(x)
    h0 = h0.at[current_node, IN].set(1.0)
    h0 = h0.at[target_node, IN + 1].set(1.0)
    h0 = h0.at[:N, ONES_COL].set(1.0)

    w1l = jnp.zeros((F0, H), f32).at[: IN + 2].set(W1l)
    w1r = jnp.zeros((F0, H), f32).at[: IN + 2].set(W1r)
    b1r = b1.reshape(1, H)
    b2r = b2.reshape(1, H)
    b3r = b3.reshape(1, H)

    wfc = jnp.zeros((H, OUTP), f32).at[:, :OUT].set(Wfc)
    bfcp = jnp.full((OUTP,), -1e30, f32).at[:OUT].set(bfc).reshape(1, OUTP)

    src = edge_index[0]
    dst = edge_index[1]

    z128 = jnp.zeros((RPT, H), f32)

    p1 = _agg(h0, src, dst, z128)
    h1, invd = _tc_layer1(p1, h0, w1l, w1r, b1r)
    p2 = _agg(h1, src, dst, z128)
    h2 = _tc_layerN(p2, h1, invd, W2l, W2r, b2r)
    p3 = _agg(h2, src, dst, z128)
    h3 = _tc_layerN(p3, h2, invd, W3l, W3r, b3r)

    logits, gmax = _tc_logits(h3, wfc, bfcp)

    pad = EPADM - E
    src_k = jnp.concatenate([src, jnp.broadcast_to(src[0], (pad,))])
    dst_k = jnp.concatenate([dst, jnp.broadcast_to(dst[0], (pad,))])

    S_flat = _sc_mask(logits.reshape(-1), gmax, src_k, dst_k)

    return _tc_renorm(S_flat)


# speedup vs baseline: 1.1540x; 1.1361x over previous
import functools

import jax
import jax.numpy as jnp
from jax import lax
from jax.experimental import pallas as pl
from jax.experimental.pallas import tpu as pltpu
from jax.experimental.pallas import tpu_sc as plsc

N = 10000
IN = 32
H = 128
OUT = N + 1
E = 160000

NC = 2
NS = 16
NW = NC * NS

NPAD = 10240
F0 = 128
ONES_COL = IN + 2
OUTP = 10240

CH = 128
NECH = E // CH
RPT = NPAD // NS

_PREC = jax.lax.Precision.HIGHEST


def _mesh():
    return plsc.VectorSubcoreMesh(
        core_axis_name="c", subcore_axis_name="s", num_cores=NC, num_subcores=NS
    )


def _make_agg(F):
    @functools.partial(
        pl.kernel,
        mesh=_mesh(),
        out_type=jax.ShapeDtypeStruct((NC, NPAD, F), jnp.float32),
        scratch_types=[
            pltpu.VMEM((CH,), jnp.int32),
            pltpu.VMEM((CH,), jnp.int32),
            pltpu.VMEM((CH, F), jnp.float32),
            pltpu.VMEM_SHARED((NPAD, F), jnp.float32),
            pltpu.SemaphoreType.DMA,
        ],
    )
    def agg(h_hbm, src_hbm, dst_hbm, zrows_hbm, out_hbm, idx_s, idx_d, rows, acc, sem):
        c = lax.axis_index("c")
        s = lax.axis_index("s")
        wid = s * NC + c
        row0 = s * RPT
        pltpu.sync_copy(zrows_hbm, acc.at[pl.ds(row0, RPT)])
        plsc.subcore_barrier()

        def chunk(k, carry):
            cid = wid + k * NW

            @pl.when(cid < NECH)
            def _():
                off = cid * CH
                pltpu.sync_copy(src_hbm.at[pl.ds(off, CH)], idx_s)
                pltpu.sync_copy(dst_hbm.at[pl.ds(off, CH)], idx_d)
                pltpu.async_copy(h_hbm.at[idx_s], rows, sem).wait()
                pltpu.sync_copy(rows, acc.at[idx_d], add=True)

            return carry

        lax.fori_loop(0, (NECH + NW - 1) // NW, chunk, 0)
        plsc.subcore_barrier()
        pltpu.sync_copy(acc.at[pl.ds(row0, RPT)], out_hbm.at[c, pl.ds(row0, RPT)])

    return agg


_agg = _make_agg(H)


_RT = 1024


def _layer1_body(p_ref, h_ref, wl_ref, wr_ref, b_ref, out_ref, invd_ref):
    tot = p_ref[0] + p_ref[1]
    deg = tot[:, ONES_COL]
    invd = 1.0 / jnp.maximum(deg, 1.0)
    mean = tot * invd[:, None]
    acc = (
        jnp.dot(mean, wl_ref[...], precision=_PREC, preferred_element_type=jnp.float32)
        + b_ref[...]
        + jnp.dot(h_ref[...], wr_ref[...], precision=_PREC, preferred_element_type=jnp.float32)
    )
    i = pl.program_id(0)
    rows = i * _RT + lax.broadcasted_iota(jnp.int32, (_RT, 1), 0)
    out_ref[...] = jnp.where(rows < N, jnp.maximum(acc, 0.0), 0.0)
    invd_ref[...] = invd


def _layerN_body(p_ref, h_ref, invd_ref, wl_ref, wr_ref, b_ref, out_ref):
    invd = invd_ref[...]
    mean = (p_ref[0] + p_ref[1]) * invd[:, None]
    acc = (
        jnp.dot(mean, wl_ref[...], precision=_PREC, preferred_element_type=jnp.float32)
        + b_ref[...]
        + jnp.dot(h_ref[...], wr_ref[...], precision=_PREC, preferred_element_type=jnp.float32)
    )
    i = pl.program_id(0)
    rows = i * _RT + lax.broadcasted_iota(jnp.int32, (_RT, 1), 0)
    out_ref[...] = jnp.where(rows < N, jnp.maximum(acc, 0.0), 0.0)


def _tc_layer1(p, h, wl, wr, b):
    F = h.shape[1]
    return pl.pallas_call(
        _layer1_body,
        grid=(NPAD // _RT,),
        in_specs=[
            pl.BlockSpec((NC, _RT, F), lambda i: (0, i, 0)),
            pl.BlockSpec((_RT, F), lambda i: (i, 0)),
            pl.BlockSpec((F, H), lambda i: (0, 0)),
            pl.BlockSpec((F, H), lambda i: (0, 0)),
            pl.BlockSpec((1, H), lambda i: (0, 0)),
        ],
        out_specs=[
            pl.BlockSpec((_RT, H), lambda i: (i, 0)),
            pl.BlockSpec((_RT,), lambda i: (i,)),
        ],
        out_shape=[
            jax.ShapeDtypeStruct((NPAD, H), jnp.float32),
            jax.ShapeDtypeStruct((NPAD,), jnp.float32),
        ],
    )(p, h, wl, wr, b)


def _tc_layerN(p, h, invd, wl, wr, b):
    return pl.pallas_call(
        _layerN_body,
        grid=(NPAD // _RT,),
        in_specs=[
            pl.BlockSpec((NC, _RT, H), lambda i: (0, i, 0)),
            pl.BlockSpec((_RT, H), lambda i: (i, 0)),
            pl.BlockSpec((_RT,), lambda i: (i,)),
            pl.BlockSpec((H, H), lambda i: (0, 0)),
            pl.BlockSpec((H, H), lambda i: (0, 0)),
            pl.BlockSpec((1, H), lambda i: (0, 0)),
        ],
        out_specs=pl.BlockSpec((_RT, H), lambda i: (i, 0)),
        out_shape=jax.ShapeDtypeStruct((NPAD, H), jnp.float32),
    )(p, h, invd, wl, wr, b)


_RL = 1024
_CL = 1280


def _logits_body(h_ref, wfc_ref, bfc_ref, logits_ref, rmax_ref):
    j = pl.program_id(1)
    l = (
        jnp.dot(h_ref[...], wfc_ref[...], precision=jax.lax.Precision.DEFAULT,
                preferred_element_type=jnp.float32)
        + bfc_ref[...]
    )
    logits_ref[...] = l
    bm = jnp.max(l, axis=1)

    @pl.when(j == 0)
    def _():
        rmax_ref[...] = bm

    @pl.when(j > 0)
    def _():
        rmax_ref[...] = jnp.maximum(rmax_ref[...], bm)


def _tc_logits(h3, wfc, bfc):
    return pl.pallas_call(
        _logits_body,
        grid=(pl.cdiv(N, _RL), OUTP // _CL),
        in_specs=[
            pl.BlockSpec((_RL, H), lambda i, j: (i, 0)),
            pl.BlockSpec((H, _CL), lambda i, j: (0, j)),
            pl.BlockSpec((1, _CL), lambda i, j: (0, j)),
        ],
        out_specs=[
            pl.BlockSpec((_RL, _CL), lambda i, j: (i, j)),
            pl.BlockSpec((_RL,), lambda i, j: (i,)),
        ],
        out_shape=[
            jax.ShapeDtypeStruct((N, OUTP), jnp.float32),
            jax.ShapeDtypeStruct((N,), jnp.float32),
        ],
    )(h3, wfc, bfc)


HROW = N // 2
HFLAT = HROW * OUTP
ZCH = 16384
NZCH = HFLAT // ZCH


NKT = (NECH + NS - 1) // NS
EPM = NKT * CH
EPADM = EPM * NS
NZT = (NZCH + NS - 1) // NS


@functools.partial(
    pl.kernel,
    mesh=_mesh(),
    out_type=jax.ShapeDtypeStruct((N * OUTP,), jnp.float32),
    scratch_types=[
        pltpu.VMEM((ZCH,), jnp.float32),
        pltpu.VMEM((CH,), jnp.int32),
        pltpu.VMEM((CH,), jnp.int32),
        pltpu.VMEM((CH,), jnp.int32),
        pltpu.VMEM((CH,), jnp.int32),
        pltpu.VMEM((CH,), jnp.float32),
        pltpu.VMEM((CH,), jnp.float32),
        pltpu.VMEM((CH,), jnp.float32),
        pltpu.SemaphoreType.DMA,
    ],
)
def _sc_mask(logits_hbm, rmax_hbm, src_hbm, dst_hbm, z_hbm,
             zb, idx_s, idx_d, flt_l, flt_z, mv, lv, vo, sem):
    c = lax.axis_index("c")
    s = lax.axis_index("s")
    zeros16 = jnp.zeros((16,), jnp.float32)

    def zfill(i, carry):
        zb[pl.ds(i * 16, 16)] = zeros16
        return carry

    with jax.named_scope("mask_zfill"):
        lax.fori_loop(0, ZCH // 16, zfill, 0)

    cbase = c * HFLAT

    def zchunk(it, carry):
        zcid = s + it * NS

        @pl.when(zcid < NZCH)
        def _():
            pltpu.sync_copy(zb, z_hbm.at[pl.ds(cbase + zcid * ZCH, ZCH)])

        return carry

    with jax.named_scope("mask_zero_hbm"):
        lax.fori_loop(0, NZT, zchunk, 0)

    plsc.subcore_barrier()

    lo = c * HROW
    hi = lo + HROW
    lane = lax.iota(jnp.int32, 16)

    def chunk(k, carry):
        cid = s + k * NS

        @pl.when(cid < NECH)
        def _():
            off = cid * CH
            pltpu.sync_copy(src_hbm.at[pl.ds(off, CH)], idx_s)
            pltpu.sync_copy(dst_hbm.at[pl.ds(off, CH)], idx_d)
            for t in range(CH // 16):
                sl = pl.ds(t * 16, 16)
                vs = idx_s[sl]
                vd = idx_d[sl]
                flt_l[sl] = vs * OUTP + vd
                inb = (vs >= lo) & (vs < hi)
                tr = lo + jax.lax.rem(off + t * 16 + lane, HROW)
                flt_z[sl] = jnp.where(inb, vs * OUTP + vd, tr * OUTP + N)
            pltpu.async_copy(rmax_hbm.at[idx_s], mv, sem).wait()
            pltpu.async_copy(logits_hbm.at[flt_l], lv, sem).wait()
            for t in range(CH // 16):
                sl = pl.ds(t * 16, 16)
                vs = idx_s[sl]
                inb = (vs >= lo) & (vs < hi)
                vo[sl] = jnp.where(inb, jnp.exp(lv[sl] - mv[sl]), 0.0)
            pltpu.async_copy(vo, z_hbm.at[flt_z], sem).wait()

        return carry

    with jax.named_scope("mask_edges"):
        lax.fori_loop(0, NKT, chunk, 0)


_R6 = 200


def _renorm_body(s_ref, out_ref):
    blk = s_ref[...].reshape(_R6, OUTP)
    rs = jnp.sum(blk, axis=1)
    out_ref[...] = (blk * (1.0 / jnp.maximum(rs, 1e-30))[:, None])[:, :OUT]


def _tc_renorm(S_flat):
    return pl.pallas_call(
        _renorm_body,
        grid=(N // _R6,),
        in_specs=[pl.BlockSpec((_R6 * OUTP,), lambda i: (i,))],
        out_specs=pl.BlockSpec((_R6, OUT), lambda i: (i, 0)),
        out_shape=jax.ShapeDtypeStruct((N, OUT), jnp.float32),
    )(S_flat)


def kernel(x, edge_index, current_node, target_node,
           W1l, b1, W1r, W2l, b2, W2r, W3l, b3, W3r, Wfc, bfc):
    f32 = jnp.float32

    h0 = jnp.zeros((NPAD, F0), f32)
    h0 = h0.at[:N, :IN].set(x)
    h0 = h0.at[current_node, IN].set(1.0)
    h0 = h0.at[target_node, IN + 1].set(1.0)
    h0 = h0.at[:N, ONES_COL].set(1.0)

    w1l = jnp.zeros((F0, H), f32).at[: IN + 2].set(W1l)
    w1r = jnp.zeros((F0, H), f32).at[: IN + 2].set(W1r)
    b1r = b1.reshape(1, H)
    b2r = b2.reshape(1, H)
    b3r = b3.reshape(1, H)

    wfc = jnp.zeros((H, OUTP), f32).at[:, :OUT].set(Wfc)
    bfcp = jnp.full((OUTP,), -1e30, f32).at[:OUT].set(bfc).reshape(1, OUTP)

    src = edge_index[0]
    dst = edge_index[1]

    z128 = jnp.zeros((RPT, H), f32)

    p1 = _agg(h0, src, dst, z128)
    h1, invd = _tc_layer1(p1, h0, w1l, w1r, b1r)
    p2 = _agg(h1, src, dst, z128)
    h2 = _tc_layerN(p2, h1, invd, W2l, W2r, b2r)
    p3 = _agg(h2, src, dst, z128)
    h3 = _tc_layerN(p3, h2, invd, W3l, W3r, b3r)

    logits, rmax = _tc_logits(h3, wfc, bfcp)

    S_flat = _sc_mask(logits.reshape(-1), rmax, src, dst)

    return _tc_renorm(S_flat)
